# baseline (speedup 1.0000x reference)
import jax
import jax.numpy as jnp
from jax import lax
from jax.experimental import pallas as pl
from jax.experimental.pallas import tpu as pltpu
from jax.experimental.pallas import tpu_sc as plsc

_DMODEL = 1024
_B = 4
_T = 2048

_NC = 2
_NS = 16
_NW = _NC * _NS
_TPW = _T // _NW
_CHUNK = 8
_NSET = _TPW // _CHUNK
_NGRP = 3
_LANES = 16
_VPR = _DMODEL // _LANES


def _emb_body(x_hbm, tok_hbm, pos_hbm, out_hbm,
              idx_v, pos_v, rows_v, gsems, wsems, psems, isem):
    wid = lax.axis_index("s") * _NC + lax.axis_index("c")
    tb = wid * _TPW

    icps = [pltpu.async_copy(x_hbm.at[b, pl.ds(tb, _TPW)],
                             idx_v.at[b], isem) for b in range(_B)]

    def start_pos(s):
        h = s % 2
        return pltpu.async_copy(
            pos_hbm.at[pl.ds(tb + s * _CHUNK, _CHUNK)], pos_v.at[h], psems[h])

    def start_gather(s, b):
        p = (s % _NGRP) * _B + b
        return pltpu.async_copy(
            tok_hbm.at[idx_v.at[b, pl.ds(s * _CHUNK, _CHUNK)]],
            rows_v[p], gsems[p])

    def start_write(s, b):
        p = (s % _NGRP) * _B + b
        base = tb + s * _CHUNK
        return pltpu.async_copy(rows_v[p],
                                out_hbm.at[b, pl.ds(base, _CHUNK)],
                                wsems[p])

    pcp = [start_pos(0), start_pos(1)]
    for cp in icps:
        cp.wait()
    g = [None] * (_NGRP * _B)
    w = [None] * (_NGRP * _B)
    for s in range(2):
        for b in range(_B):
            g[s * _B + b] = start_gather(s, b)

    for s in range(_NSET):
        pi = (s % _NGRP) * _B
        h = s % 2
        with jax.named_scope("gwait"):
            for b in range(_B):
                g[pi + b].wait()
            pcp[h].wait()

        bufs = [rows_v[pi + b] for b in range(_B)]

        def add_col(j, carry, h=h, bufs=bufs):
            for u in range(2):
                col = pl.ds(j * (2 * _LANES) + u * _LANES, _LANES)
                for r in range(_CHUNK):
                    pv = pos_v[h, r, col]
                    for buf in bufs[:2]:
                        buf[r, col] = buf[r, col] + pv
                    for buf in bufs[2:]:
                        plsc.addupdate(buf.at[r, col], pv)
            return carry

        with jax.named_scope("add"):
            lax.fori_loop(0, _VPR // 2, add_col, 0)
        for b in range(_B):
            w[pi + b] = start_write(s, b)

        if s + 2 < _NSET:
            pcp[h] = start_pos(s + 2)
            qi = ((s + 2) % _NGRP) * _B
            with jax.named_scope("wwait"):
                for b in range(_B):
                    if w[qi + b] is not None:
                        w[qi + b].wait()
            for b in range(_B):
                g[qi + b] = start_gather(s + 2, b)

    for p in range(_NGRP * _B):
        if w[p] is not None:
            w[p].wait()


@jax.jit
def kernel(x, tokemb, posemb):
    b, t = x.shape
    mesh = plsc.VectorSubcoreMesh(core_axis_name="c", subcore_axis_name="s")
    out = pl.kernel(
        _emb_body,
        out_type=jax.ShapeDtypeStruct((b, t, _DMODEL), jnp.float32),
        mesh=mesh,
        scratch_types=[
            pltpu.VMEM((_B, _TPW), jnp.int32),
            pltpu.VMEM((2, _CHUNK, _DMODEL), jnp.float32),
            [pltpu.VMEM((_CHUNK, _DMODEL), jnp.float32)] * (_NGRP * _B),
            [pltpu.SemaphoreType.DMA] * (_NGRP * _B),
            [pltpu.SemaphoreType.DMA] * (_NGRP * _B),
            [pltpu.SemaphoreType.DMA] * 2,
            pltpu.SemaphoreType.DMA,
        ],
    )(x.astype(jnp.int32), tokemb, posemb)
    return out


# speedup vs baseline: 1.0593x; 1.0593x over previous
import jax
import jax.numpy as jnp
from jax import lax
from jax.experimental import pallas as pl
from jax.experimental.pallas import tpu as pltpu
from jax.experimental.pallas import tpu_sc as plsc

_DMODEL = 1024
_B = 4
_T = 2048

_NC = 2
_NS = 16
_NW = _NC * _NS
_TPW = _T // _NW
_CHUNK = 8
_NSET = _TPW // _CHUNK
_NGRP = 3
_LANES = 16
_VPR = _DMODEL // _LANES


def _emb_body(x_hbm, tok_hbm, pos_hbm, out_hbm,
              idx_v, pos_v, rows_v, gsems, wsems, psems, isem):
    wid = lax.axis_index("s") * _NC + lax.axis_index("c")
    tb = wid * _TPW

    icps = [pltpu.async_copy(x_hbm.at[b, pl.ds(tb, _TPW)],
                             idx_v.at[b], isem) for b in range(_B)]

    def start_pos(s):
        h = s % 2
        return pltpu.async_copy(
            pos_hbm.at[pl.ds(tb + s * _CHUNK, _CHUNK)], pos_v.at[h], psems[h])

    def start_gather(s, b):
        p = (s % _NGRP) * _B + b
        return pltpu.async_copy(
            tok_hbm.at[idx_v.at[b, pl.ds(s * _CHUNK, _CHUNK)]],
            rows_v[p], gsems[p])

    def start_write(s, b):
        p = (s % _NGRP) * _B + b
        base = tb + s * _CHUNK
        return pltpu.async_copy(rows_v[p],
                                out_hbm.at[b, pl.ds(base, _CHUNK)],
                                wsems[p])

    pcp = [start_pos(0), start_pos(1)]
    for cp in icps:
        cp.wait()
    g = [None] * (_NGRP * _B)
    w = [None] * (_NGRP * _B)
    for s in range(2):
        for b in range(_B):
            g[s * _B + b] = start_gather(s, b)

    for s in range(_NSET):
        pi = (s % _NGRP) * _B
        h = s % 2
        with jax.named_scope("gwait"):
            for b in range(_B):
                g[pi + b].wait()
            pcp[h].wait()

        bufs = [rows_v[pi + b] for b in range(_B)]

        def add_col(j, carry, h=h, bufs=bufs):
            col = pl.ds(j * _LANES, _LANES)
            for r in range(_CHUNK):
                pv = pos_v[h, r, col]
                for buf in bufs[:1]:
                    buf[r, col] = buf[r, col] + pv
                for buf in bufs[1:]:
                    plsc.addupdate(buf.at[r, col], pv)
            return carry

        with jax.named_scope("add"):
            lax.fori_loop(0, _VPR, add_col, 0)
        for b in range(_B):
            w[pi + b] = start_write(s, b)

        if s + 2 < _NSET:
            pcp[h] = start_pos(s + 2)
            qi = ((s + 2) % _NGRP) * _B
            with jax.named_scope("wwait"):
                for b in range(_B):
                    if w[qi + b] is not None:
                        w[qi + b].wait()
            for b in range(_B):
                g[qi + b] = start_gather(s + 2, b)

    for p in range(_NGRP * _B):
        if w[p] is not None:
            w[p].wait()


@jax.jit
def kernel(x, tokemb, posemb):
    b, t = x.shape
    mesh = plsc.VectorSubcoreMesh(core_axis_name="c", subcore_axis_name="s")
    out = pl.kernel(
        _emb_body,
        out_type=jax.ShapeDtypeStruct((b, t, _DMODEL), jnp.float32),
        mesh=mesh,
        scratch_types=[
            pltpu.VMEM((_B, _TPW), jnp.int32),
            pltpu.VMEM((2, _CHUNK, _DMODEL), jnp.float32),
            [pltpu.VMEM((_CHUNK, _DMODEL), jnp.float32)] * (_NGRP * _B),
            [pltpu.SemaphoreType.DMA] * (_NGRP * _B),
            [pltpu.SemaphoreType.DMA] * (_NGRP * _B),
            [pltpu.SemaphoreType.DMA] * 2,
            pltpu.SemaphoreType.DMA,
        ],
    )(x.astype(jnp.int32), tokemb, posemb)
    return out
